# Initial kernel scaffold; baseline (speedup 1.0000x reference)
import jax
import jax.numpy as jnp
from jax.experimental import pallas as pl


def kernel(x, in_proj_w, conv_w, conv_b, x_proj_w, dt_proj_w, dt_proj_b, A_log, D, out_proj_w):
    raise NotImplementedError("write your pallas kernel here")


# speedup vs baseline: 7.4338x; 7.4338x over previous
import functools

import jax
import jax.numpy as jnp
from jax.experimental import pallas as pl
from jax.experimental.pallas import tpu as pltpu

_DM = 128
_DIN = 256
_N = 16
_RANK = 8
_NLAYER = 4
_T = 128


def _silu(v):
    return v * jax.nn.sigmoid(v)


def _mamba_kernel(x_ref, inw_ref, cw_ref, cb_ref, xpw_ref, dtw_ref, dtb_ref,
                  a_ref, d_ref, ow_ref, o_ref,
                  xa_pre, xa_s, z_s, dt_s, w_s, pj_s, g3, w3, xn_s,
                  *, nc):
    _NC = nc
    f32 = jnp.float32
    xa_pre[0:8, :] = jnp.zeros((8, _DIN), f32)

    for l in range(_NLAYER):
        inw = inw_ref[l]
        cwv = cw_ref[l]
        cbv = cb_ref[l]
        xpw = xpw_ref[l]
        dtw = dtw_ref[l]
        dtb = dtb_ref[l]
        av = a_ref[l]
        dv = d_ref[l]
        oww = ow_ref[l]

        def pass1(ci, _, l=l, inw=inw):
            c0 = pl.multiple_of(ci * _T, _T)
            rs = pl.ds(c0, _T)
            if l == 0:
                xin = x_ref[0, rs, :]
            else:
                xin = xn_s[rs, :]
            xz = jnp.dot(xin, inw, preferred_element_type=f32)
            xa_pre[pl.ds(pl.multiple_of(c0 + 8, 8), _T), :] = xz[:, :_DIN]
            z_s[rs, :] = xz[:, _DIN:]
            return 0

        jax.lax.fori_loop(0, _NC, pass1, 0)

        def pass2(ci, _, cwv=cwv, cbv=cbv, xpw=xpw, dtw=dtw, dtb=dtb):
            c0 = pl.multiple_of(ci * _T, _T)
            rs = pl.ds(c0, _T)
            win = xa_pre[pl.ds(c0, _T + 8), :]
            conv = cbv + win[5:5 + _T, :] * cwv[0:1, :]
            for k in range(1, 4):
                conv = conv + win[5 + k:5 + k + _T, :] * cwv[k:k + 1, :]
            xa_c = _silu(conv)
            xa_s[rs, :] = xa_c
            pj = jnp.dot(xa_c, xpw, preferred_element_type=f32)
            pj_s[rs, :] = pj
            dtr = jnp.dot(pj[:, :_RANK], dtw, preferred_element_type=f32) + dtb
            dt_c = jax.nn.softplus(dtr)
            dt_s[rs, :] = dt_c
            w_s[rs, :] = dt_c * xa_c
            return 0

        jax.lax.fori_loop(0, _NC, pass2, 0)

        def pass3(ci, h, l=l, av=av, dv=dv, oww=oww):
            c0 = pl.multiple_of(ci * _T, _T)
            rs = pl.ds(c0, _T)
            dt_c = dt_s[rs, :]
            for n in range(_N):
                g3[:, n, :] = jnp.exp(dt_c * av[n:n + 1, :])
            wv = w_s[rs, :]
            pj = pj_s[rs, :]
            bv = pj[:, _RANK:_RANK + _N]
            cv = pj[:, _RANK + _N:_RANK + 2 * _N]
            for n in range(_N):
                w3[:, n, :] = wv * bv[:, n:n + 1]

            def step(t, hh):
                hh = g3[t] * hh + w3[t]
                w3[t] = hh
                return hh

            h = jax.lax.fori_loop(0, _T, step, h)

            yv = w3[:, 0, :] * cv[:, 0:1]
            for n in range(1, _N):
                yv = yv + w3[:, n, :] * cv[:, n:n + 1]
            zv = z_s[rs, :]
            y_c = (yv + xa_s[rs, :] * dv) * _silu(zv)
            oc = jnp.dot(y_c, oww, preferred_element_type=f32)
            if l == _NLAYER - 1:
                o_ref[0, rs, :] = oc
            else:
                xn_s[rs, :] = oc
            return h

        h0 = jnp.zeros((_N, _DIN), f32)
        jax.lax.fori_loop(0, _NC, pass3, h0)


@jax.jit
def kernel(x, in_proj_w, conv_w, conv_b, x_proj_w, dt_proj_w, dt_proj_b,
           A_log, D, out_proj_w):
    f32 = jnp.float32
    B, _L, _ = x.shape
    inw_t = in_proj_w.transpose(0, 2, 1)
    cw_t = conv_w.transpose(0, 2, 1)
    cb2 = conv_b[:, None, :]
    xpw_t = x_proj_w.transpose(0, 2, 1)
    dtw_t = dt_proj_w.transpose(0, 2, 1)
    dtb2 = dt_proj_b[:, None, :]
    a_t = (-jnp.exp(A_log)).transpose(0, 2, 1)
    d2 = D[:, None, :]
    ow_t = out_proj_w.transpose(0, 2, 1)

    full = lambda s: pl.BlockSpec(s, lambda b: (0,) * len(s))
    return pl.pallas_call(
        functools.partial(_mamba_kernel, nc=_L // _T),
        grid=(B,),
        in_specs=[
            pl.BlockSpec((1, _L, _DM), lambda b: (b, 0, 0)),
            full(inw_t.shape), full(cw_t.shape), full(cb2.shape),
            full(xpw_t.shape), full(dtw_t.shape), full(dtb2.shape),
            full(a_t.shape), full(d2.shape), full(ow_t.shape),
        ],
        out_specs=pl.BlockSpec((1, _L, _DM), lambda b: (b, 0, 0)),
        out_shape=jax.ShapeDtypeStruct((B, _L, _DM), f32),
        scratch_shapes=[
            pltpu.VMEM((_L + 8, _DIN), f32),
            pltpu.VMEM((_L, _DIN), f32),
            pltpu.VMEM((_L, _DIN), f32),
            pltpu.VMEM((_L, _DIN), f32),
            pltpu.VMEM((_L, _DIN), f32),
            pltpu.VMEM((_L, 2 * _N + _RANK), f32),
            pltpu.VMEM((_T, _N, _DIN), f32),
            pltpu.VMEM((_T, _N, _DIN), f32),
            pltpu.VMEM((_L, _DM), f32),
        ],
        compiler_params=pltpu.CompilerParams(
            dimension_semantics=(pltpu.PARALLEL,),
            vmem_limit_bytes=56 * 1024 * 1024,
        ),
    )(x, inw_t, cw_t, cb2, xpw_t, dtw_t, dtb2, a_t, d2, ow_t)
